# baseline (speedup 1.0000x reference)
import functools

import jax
import jax.numpy as jnp
from jax.experimental import pallas as pl
from jax.experimental.pallas import tpu as pltpu

_TOP_K = 8
_N_EXPERTS = 64
_N_GROUP = 8
_GROUP_SIZE = _N_EXPERTS // _N_GROUP
_TOPK_GROUP = 4
_SCALE = 2.5
_HIDDEN = 768

_NEG = float("-inf")


def _roll_up(x, s):
    return jnp.roll(x, -s, axis=0)


def _router_block(hs_hbm, w_ref, b_ref, idx_ref, wgt_ref, buf, sems, *,
                  block_tokens):
    B = block_tokens
    i = pl.program_id(0)
    nb = pl.num_programs(0)

    def copy_in(slot, blk):
        return pltpu.make_async_copy(
            hs_hbm.at[pl.ds(blk * B, B), :], buf.at[slot], sems.at[slot])

    @pl.when(i == 0)
    def _():
        copy_in(0, 0).start()

    @pl.when(i + 1 < nb)
    def _():
        copy_in((i + 1) % 2, i + 1).start()

    copy_in(i % 2, i).wait()

    hb = pltpu.bitcast(buf[i % 2], jnp.int16)
    hi = hb.astype(jnp.int32)
    hs = jax.lax.bitcast_convert_type((hi << 13) & jnp.int32(-1879056384),
                                      jnp.float32)
    w = w_ref[...]
    logits = jax.lax.dot_general(
        w, hs, (((1,), (1,)), ((), ())),
        preferred_element_type=jnp.float32,
    )
    scores = jax.nn.sigmoid(logits)
    sfc = scores + b_ref[...]

    sfc_c = [sfc[c * 8:(c + 1) * 8, :] for c in range(_N_GROUP)]
    sco_c = [scores[c * 8:(c + 1) * 8, :] for c in range(_N_GROUP)]

    m1s = list(sfc_c)
    m2s = [jnp.full((_GROUP_SIZE, B), _NEG, jnp.float32)] * _N_GROUP
    while len(m1s) > 1:
        n1, n2 = [], []
        for j in range(0, len(m1s), 2):
            a1, a2, b1, b2 = m1s[j], m2s[j], m1s[j + 1], m2s[j + 1]
            n1.append(jnp.maximum(a1, b1))
            n2.append(jnp.maximum(jnp.minimum(a1, b1), jnp.maximum(a2, b2)))
        m1s, m2s = n1, n2
    gs = m1s[0] + m2s[0]

    ii8 = jax.lax.broadcasted_iota(jnp.int32, (_N_GROUP, B), 0)
    gmask = jnp.zeros((_N_GROUP, B), dtype=jnp.bool_)
    work_g = gs
    for _ in range(_TOPK_GROUP):
        k8, p8 = work_g, ii8
        for s in (4, 2, 1):
            rk, rp = _roll_up(k8, s), _roll_up(p8, s)
            cond = k8 >= rk
            k8 = jnp.maximum(k8, rk)
            p8 = jnp.where(cond, p8, rp)
        gwin = jnp.broadcast_to(p8[0:1, :], (_N_GROUP, B))
        sel = ii8 == gwin
        gmask = jnp.logical_or(gmask, sel)
        work_g = jnp.where(sel, _NEG, work_g)

    row_iota = jax.lax.broadcasted_iota(jnp.int32, (_GROUP_SIZE, B), 0)
    kc, pc = [], []
    for c in range(_N_GROUP):
        kc.append(jnp.where(gmask, sfc_c[c], jnp.float32(0.0)))
        sb = jax.lax.bitcast_convert_type(sco_c[c], jnp.int32)
        pc.append((sb & jnp.int32(~0x3F)) | (row_iota * 8 + c))

    pwin_rows = []
    for t in range(_TOP_K):
        ks, ps = list(kc), list(pc)
        while len(ks) > 1:
            nk, np_ = [], []
            for j in range(0, len(ks), 2):
                cond = ks[j] >= ks[j + 1]
                nk.append(jnp.maximum(ks[j], ks[j + 1]))
                np_.append(jnp.where(cond, ps[j], ps[j + 1]))
            ks, ps = nk, np_
        k8, p8 = ks[0], ps[0]
        for s in (4, 2, 1):
            rk, rp = _roll_up(k8, s), _roll_up(p8, s)
            cond = k8 >= rk
            k8 = jnp.maximum(k8, rk)
            p8 = jnp.where(cond, p8, rp)
        pwin = p8[0:1, :]
        pwin_rows.append(pwin)
        if t != _TOP_K - 1:
            pb = jnp.broadcast_to(pwin, (_GROUP_SIZE, B))
            for c in range(_N_GROUP):
                kc[c] = jnp.where(pc[c] == pb, _NEG, kc[c])

    pall = jnp.concatenate(pwin_rows, axis=0)
    topk_idx = pall & 0x3F
    topk_w = jax.lax.bitcast_convert_type(pall & jnp.int32(~0x3F),
                                          jnp.float32)
    topk_w = topk_w / (jnp.sum(topk_w, axis=0, keepdims=True)
                       + jnp.float32(1e-20))
    topk_w = topk_w * jnp.float32(_SCALE)

    idx_ref[...] = topk_idx
    wgt_ref[...] = topk_w


def kernel(hidden_states, weight, e_score_correction_bias):
    n = hidden_states.shape[0]
    block_tokens = 4096
    grid = (n // block_tokens,)
    perm = (jnp.arange(_N_EXPERTS) % 8) * 8 + jnp.arange(_N_EXPERTS) // 8
    bias = e_score_correction_bias.astype(jnp.float32)[perm]
    bias = bias.reshape(_N_EXPERTS, 1)
    w32 = weight.astype(jnp.float32)[perm, :] * jnp.float32(2.0 ** 112)
    body = functools.partial(_router_block, block_tokens=block_tokens)
    topk_idx, topk_w = pl.pallas_call(
        body,
        grid=grid,
        in_specs=[
            pl.BlockSpec(memory_space=pl.ANY),
            pl.BlockSpec((_N_EXPERTS, _HIDDEN), lambda i: (0, 0)),
            pl.BlockSpec((_N_EXPERTS, 1), lambda i: (0, 0)),
        ],
        scratch_shapes=[
            pltpu.VMEM((2, block_tokens, _HIDDEN), jnp.bfloat16),
            pltpu.SemaphoreType.DMA((2,)),
        ],
        out_specs=[
            pl.BlockSpec((_TOP_K, block_tokens), lambda i: (0, i)),
            pl.BlockSpec((_TOP_K, block_tokens), lambda i: (0, i)),
        ],
        out_shape=[
            jax.ShapeDtypeStruct((_TOP_K, n), jnp.int32),
            jax.ShapeDtypeStruct((_TOP_K, n), jnp.float32),
        ],
        compiler_params=pltpu.CompilerParams(
            dimension_semantics=("arbitrary",),
        ),
    )(jax.lax.bitcast_convert_type(hidden_states, jnp.bfloat16), w32, bias)
    return topk_idx.T, topk_w.T


# speedup vs baseline: 1.0089x; 1.0089x over previous
import functools

import jax
import jax.numpy as jnp
from jax.experimental import pallas as pl
from jax.experimental.pallas import tpu as pltpu

_TOP_K = 8
_N_EXPERTS = 64
_N_GROUP = 8
_GROUP_SIZE = _N_EXPERTS // _N_GROUP
_TOPK_GROUP = 4
_SCALE = 2.5
_HIDDEN = 768

_NEG = float("-inf")


def _roll_up(x, s):
    return jnp.roll(x, -s, axis=0)


def _router_block(hs_ref, w_ref, b_ref, idx_ref, wgt_ref, *, block_tokens):
    B = block_tokens
    hb = pltpu.bitcast(hs_ref[...], jnp.int16).reshape(B, _HIDDEN)
    hi = hb.astype(jnp.int32)
    hs = jax.lax.bitcast_convert_type((hi << 13) & jnp.int32(-1879056384),
                                      jnp.float32)
    w = w_ref[...]
    logits = jax.lax.dot_general(
        w, hs, (((1,), (1,)), ((), ())),
        preferred_element_type=jnp.float32,
    )
    scores = jax.nn.sigmoid(logits)
    sfc = scores + b_ref[...]

    sfc_c = [sfc[c * 8:(c + 1) * 8, :] for c in range(_N_GROUP)]
    sco_c = [scores[c * 8:(c + 1) * 8, :] for c in range(_N_GROUP)]

    m1s = list(sfc_c)
    m2s = [jnp.full((_GROUP_SIZE, B), _NEG, jnp.float32)] * _N_GROUP
    while len(m1s) > 1:
        n1, n2 = [], []
        for j in range(0, len(m1s), 2):
            a1, a2, b1, b2 = m1s[j], m2s[j], m1s[j + 1], m2s[j + 1]
            n1.append(jnp.maximum(a1, b1))
            n2.append(jnp.maximum(jnp.minimum(a1, b1), jnp.maximum(a2, b2)))
        m1s, m2s = n1, n2
    gs = m1s[0] + m2s[0]

    ii8 = jax.lax.broadcasted_iota(jnp.int32, (_N_GROUP, B), 0)
    gmask = jnp.zeros((_N_GROUP, B), dtype=jnp.bool_)
    work_g = gs
    for _ in range(_TOPK_GROUP):
        k8, p8 = work_g, ii8
        for s in (4, 2, 1):
            rk, rp = _roll_up(k8, s), _roll_up(p8, s)
            cond = k8 >= rk
            k8 = jnp.maximum(k8, rk)
            p8 = jnp.where(cond, p8, rp)
        gwin = jnp.broadcast_to(p8[0:1, :], (_N_GROUP, B))
        sel = ii8 == gwin
        gmask = jnp.logical_or(gmask, sel)
        work_g = jnp.where(sel, _NEG, work_g)

    row_iota = jax.lax.broadcasted_iota(jnp.int32, (_GROUP_SIZE, B), 0)
    kc, pc = [], []
    for c in range(_N_GROUP):
        kc.append(jnp.where(gmask, sfc_c[c], jnp.float32(0.0)))
        sb = jax.lax.bitcast_convert_type(sco_c[c], jnp.int32)
        pc.append((sb & jnp.int32(~0x3F)) | (row_iota * 8 + c))

    pwin_rows = []
    for t in range(_TOP_K):
        ks, ps = list(kc), list(pc)
        while len(ks) > 1:
            nk, np_ = [], []
            for j in range(0, len(ks), 2):
                cond = ks[j] >= ks[j + 1]
                nk.append(jnp.maximum(ks[j], ks[j + 1]))
                np_.append(jnp.where(cond, ps[j], ps[j + 1]))
            ks, ps = nk, np_
        k8, p8 = ks[0], ps[0]
        for s in (4, 2, 1):
            rk, rp = _roll_up(k8, s), _roll_up(p8, s)
            cond = k8 >= rk
            k8 = jnp.maximum(k8, rk)
            p8 = jnp.where(cond, p8, rp)
        pwin = p8[0:1, :]
        pwin_rows.append(pwin)
        if t != _TOP_K - 1:
            pb = jnp.broadcast_to(pwin, (_GROUP_SIZE, B))
            for c in range(_N_GROUP):
                kc[c] = jnp.where(pc[c] == pb, _NEG, kc[c])

    pall = jnp.concatenate(pwin_rows, axis=0)
    topk_idx = pall & 0x3F
    topk_w = jax.lax.bitcast_convert_type(pall & jnp.int32(~0x3F),
                                          jnp.float32)
    topk_w = topk_w / (jnp.sum(topk_w, axis=0, keepdims=True)
                       + jnp.float32(1e-20))
    topk_w = topk_w * jnp.float32(_SCALE)

    idx_ref[...] = topk_idx
    wgt_ref[...] = topk_w


def kernel(hidden_states, weight, e_score_correction_bias):
    n = hidden_states.shape[0]
    block_tokens = 4096
    grid = (n // block_tokens,)
    perm = (jnp.arange(_N_EXPERTS) % 8) * 8 + jnp.arange(_N_EXPERTS) // 8
    bias = e_score_correction_bias.astype(jnp.float32)[perm]
    bias = bias.reshape(_N_EXPERTS, 1)
    w32 = weight.astype(jnp.float32)[perm, :] * jnp.float32(2.0 ** 112)
    body = functools.partial(_router_block, block_tokens=block_tokens)
    topk_idx, topk_w = pl.pallas_call(
        body,
        grid=grid,
        in_specs=[
            pl.BlockSpec((block_tokens // 8, 8, _HIDDEN),
                         lambda i: (i, 0, 0)),
            pl.BlockSpec((_N_EXPERTS, _HIDDEN), lambda i: (0, 0)),
            pl.BlockSpec((_N_EXPERTS, 1), lambda i: (0, 0)),
        ],
        out_specs=[
            pl.BlockSpec((_TOP_K, block_tokens), lambda i: (0, i)),
            pl.BlockSpec((_TOP_K, block_tokens), lambda i: (0, i)),
        ],
        out_shape=[
            jax.ShapeDtypeStruct((_TOP_K, n), jnp.int32),
            jax.ShapeDtypeStruct((_TOP_K, n), jnp.float32),
        ],
        compiler_params=pltpu.CompilerParams(
            dimension_semantics=("arbitrary",),
        ),
    )(jax.lax.bitcast_convert_type(hidden_states,
                                   jnp.bfloat16).reshape(n // 8, 8, _HIDDEN),
      w32, bias)
    return topk_idx.T, topk_w.T


# speedup vs baseline: 1.0172x; 1.0082x over previous
import functools

import jax
import jax.numpy as jnp
from jax.experimental import pallas as pl
from jax.experimental.pallas import tpu as pltpu

_TOP_K = 8
_N_EXPERTS = 64
_N_GROUP = 8
_GROUP_SIZE = _N_EXPERTS // _N_GROUP
_TOPK_GROUP = 4
_SCALE = 2.5
_HIDDEN = 768

_NEG = float("-inf")


def _roll_up(x, s):
    return jnp.roll(x, -s, axis=0)


def _router_block(hs_ref, w_ref, b_ref, idx_ref, wgt_ref, *, block_tokens):
    B = block_tokens
    hb = pltpu.bitcast(hs_ref[...], jnp.int16).reshape(B, _HIDDEN)
    hi = hb.astype(jnp.int32)
    hs = jax.lax.bitcast_convert_type((hi << 13) & jnp.int32(-1879056384),
                                      jnp.float32)
    w = w_ref[...]
    logits = jax.lax.dot_general(
        w, hs, (((1,), (1,)), ((), ())),
        preferred_element_type=jnp.float32,
    )
    scores = 0.5 * jnp.tanh(0.5 * logits) + 0.5
    sfc = scores + b_ref[...]

    sfc_c = [sfc[c * 8:(c + 1) * 8, :] for c in range(_N_GROUP)]
    sco_c = [scores[c * 8:(c + 1) * 8, :] for c in range(_N_GROUP)]

    m1s = list(sfc_c)
    m2s = [jnp.full((_GROUP_SIZE, B), _NEG, jnp.float32)] * _N_GROUP
    while len(m1s) > 1:
        n1, n2 = [], []
        for j in range(0, len(m1s), 2):
            a1, a2, b1, b2 = m1s[j], m2s[j], m1s[j + 1], m2s[j + 1]
            n1.append(jnp.maximum(a1, b1))
            n2.append(jnp.maximum(jnp.minimum(a1, b1), jnp.maximum(a2, b2)))
        m1s, m2s = n1, n2
    gs = m1s[0] + m2s[0]

    ii8 = jax.lax.broadcasted_iota(jnp.int32, (_N_GROUP, B), 0)
    gmask = jnp.zeros((_N_GROUP, B), dtype=jnp.bool_)
    work_g = gs
    for _ in range(_TOPK_GROUP):
        k8, p8 = work_g, ii8
        for s in (4, 2, 1):
            rk, rp = _roll_up(k8, s), _roll_up(p8, s)
            cond = k8 >= rk
            k8 = jnp.maximum(k8, rk)
            p8 = jnp.where(cond, p8, rp)
        gwin = jnp.broadcast_to(p8[0:1, :], (_N_GROUP, B))
        sel = ii8 == gwin
        gmask = jnp.logical_or(gmask, sel)
        work_g = jnp.where(sel, _NEG, work_g)

    row_iota = jax.lax.broadcasted_iota(jnp.int32, (_GROUP_SIZE, B), 0)
    kc, pc = [], []
    for c in range(_N_GROUP):
        kc.append(jnp.where(gmask, sfc_c[c], jnp.float32(0.0)))
        sb = jax.lax.bitcast_convert_type(sco_c[c], jnp.int32)
        pc.append((sb & jnp.int32(~0x3F)) | (row_iota * 8 + c))

    pwin_rows = []
    for t in range(_TOP_K):
        ks, ps = list(kc), list(pc)
        while len(ks) > 1:
            nk, np_ = [], []
            for j in range(0, len(ks), 2):
                cond = ks[j] >= ks[j + 1]
                nk.append(jnp.maximum(ks[j], ks[j + 1]))
                np_.append(jnp.where(cond, ps[j], ps[j + 1]))
            ks, ps = nk, np_
        k8, p8 = ks[0], ps[0]
        for s in (4, 2, 1):
            rk, rp = _roll_up(k8, s), _roll_up(p8, s)
            cond = k8 >= rk
            k8 = jnp.maximum(k8, rk)
            p8 = jnp.where(cond, p8, rp)
        pwin = p8[0:1, :]
        pwin_rows.append(pwin)
        if t != _TOP_K - 1:
            pb = jnp.broadcast_to(pwin, (_GROUP_SIZE, B))
            for c in range(_N_GROUP):
                kc[c] = jnp.where(pc[c] == pb, _NEG, kc[c])

    pall = jnp.concatenate(pwin_rows, axis=0)
    topk_idx = pall & 0x3F
    topk_w = jax.lax.bitcast_convert_type(pall & jnp.int32(~0x3F),
                                          jnp.float32)
    topk_w = topk_w / (jnp.sum(topk_w, axis=0, keepdims=True)
                       + jnp.float32(1e-20))
    topk_w = topk_w * jnp.float32(_SCALE)

    idx_ref[...] = topk_idx
    wgt_ref[...] = topk_w


def kernel(hidden_states, weight, e_score_correction_bias):
    n = hidden_states.shape[0]
    block_tokens = 4096
    grid = (n // block_tokens,)
    perm = (jnp.arange(_N_EXPERTS) % 8) * 8 + jnp.arange(_N_EXPERTS) // 8
    bias = e_score_correction_bias.astype(jnp.float32)[perm]
    bias = bias.reshape(_N_EXPERTS, 1)
    w32 = weight.astype(jnp.float32)[perm, :] * jnp.float32(2.0 ** 112)
    body = functools.partial(_router_block, block_tokens=block_tokens)
    topk_idx, topk_w = pl.pallas_call(
        body,
        grid=grid,
        in_specs=[
            pl.BlockSpec((block_tokens // 8, 8, _HIDDEN),
                         lambda i: (i, 0, 0)),
            pl.BlockSpec((_N_EXPERTS, _HIDDEN), lambda i: (0, 0)),
            pl.BlockSpec((_N_EXPERTS, 1), lambda i: (0, 0)),
        ],
        out_specs=[
            pl.BlockSpec((_TOP_K, block_tokens), lambda i: (0, i)),
            pl.BlockSpec((_TOP_K, block_tokens), lambda i: (0, i)),
        ],
        out_shape=[
            jax.ShapeDtypeStruct((_TOP_K, n), jnp.int32),
            jax.ShapeDtypeStruct((_TOP_K, n), jnp.float32),
        ],
        compiler_params=pltpu.CompilerParams(
            dimension_semantics=("arbitrary",),
        ),
    )(jax.lax.bitcast_convert_type(hidden_states,
                                   jnp.bfloat16).reshape(n // 8, 8, _HIDDEN),
      w32, bias)
    return topk_idx.T, topk_w.T
